# Initial kernel scaffold; baseline (speedup 1.0000x reference)
import jax
import jax.numpy as jnp
from jax.experimental import pallas as pl


def kernel(adj, x, W_gcn, b_gcn, W_lin, gamma, beta):
    raise NotImplementedError("write your pallas kernel here")


# speedup vs baseline: 25.8718x; 25.8718x over previous
import functools

import jax
import jax.numpy as jnp
from jax import lax
from jax.experimental import pallas as pl
from jax.experimental.pallas import tpu as pltpu
from jax.experimental.pallas import tpu_sc as plsc

N = 10000
E = 320000
D = 128

NC = 2
NS = 16
NW = NC * NS

CHUNK = 80
EPW = E // NW
ROWS_PER_LOAD = 25
N_CHUNKS = EPW // CHUNK
N_LOADS = N_CHUNKS // ROWS_PER_LOAD

NPAD = 10240
PER_TILE = NPAD // NS

_mesh = plsc.VectorSubcoreMesh(core_axis_name="c", subcore_axis_name="s")


@functools.partial(
    pl.kernel,
    out_type=jax.ShapeDtypeStruct((NC, 1, NPAD), jnp.float32),
    mesh=_mesh,
    scratch_types=[
        pltpu.VMEM((ROWS_PER_LOAD, 1, CHUNK), jnp.int32),
        pltpu.VMEM((CHUNK,), jnp.float32),
        pltpu.VMEM_SHARED((NPAD,), jnp.float32),
    ],
)
def _deg_kernel(col_hbm, zeros_hbm, ones_hbm, out_hbm, idx_v, ones_v, hist_sp):
    c = lax.axis_index("c")
    s = lax.axis_index("s")
    wid = c * NS + s
    pltpu.sync_copy(zeros_hbm, hist_sp.at[pl.ds(s * PER_TILE, PER_TILE)])
    pltpu.sync_copy(ones_hbm, ones_v)
    plsc.subcore_barrier()

    def outer(o, _):
        pltpu.sync_copy(col_hbm.at[wid * N_LOADS + o], idx_v)

        def inner(j, _):
            pltpu.sync_copy(ones_v, hist_sp.at[idx_v.at[j, 0]], add=True)
            return 0

        lax.fori_loop(0, ROWS_PER_LOAD, inner, 0)
        return 0

    lax.fori_loop(0, N_LOADS, outer, 0)
    plsc.subcore_barrier()
    pltpu.sync_copy(
        hist_sp.at[pl.ds(s * PER_TILE, PER_TILE)],
        out_hbm.at[c, 0, pl.ds(s * PER_TILE, PER_TILE)],
    )


@functools.partial(
    pl.kernel,
    out_type=jax.ShapeDtypeStruct((NC, NPAD, D), jnp.float32),
    mesh=_mesh,
    scratch_types=[
        pltpu.VMEM((ROWS_PER_LOAD, 1, CHUNK), jnp.int32),
        pltpu.VMEM((ROWS_PER_LOAD, 1, CHUNK), jnp.int32),
        pltpu.VMEM((CHUNK, D), jnp.float32),
        pltpu.VMEM_SHARED((NPAD, D), jnp.float32),
        pltpu.SemaphoreType.DMA,
    ],
)
def _edge_kernel(row_hbm, col_hbm, hp_hbm, zeros_hbm, out_hbm,
                 idxr_v, idxc_v, rows_v, acc_sp, sem):
    c = lax.axis_index("c")
    s = lax.axis_index("s")
    wid = c * NS + s
    pltpu.sync_copy(zeros_hbm, acc_sp.at[pl.ds(s * PER_TILE, PER_TILE)])
    plsc.subcore_barrier()

    def outer(o, _):
        base = wid * N_LOADS + o
        pltpu.sync_copy(row_hbm.at[base], idxr_v)
        pltpu.sync_copy(col_hbm.at[base], idxc_v)

        def inner(j, _):
            pltpu.async_copy(hp_hbm.at[idxr_v.at[j, 0]], rows_v, sem).wait()
            pltpu.sync_copy(rows_v, acc_sp.at[idxc_v.at[j, 0]], add=True)
            return 0

        lax.fori_loop(0, ROWS_PER_LOAD, inner, 0)
        return 0

    lax.fori_loop(0, N_LOADS, outer, 0)
    plsc.subcore_barrier()
    pltpu.sync_copy(
        acc_sp.at[pl.ds(s * PER_TILE, PER_TILE)],
        out_hbm.at[c, pl.ds(s * PER_TILE, PER_TILE)],
    )


BLK = 1000
GRID = N // BLK


def _prep_body(x_ref, wg_ref, wl_ref, degp_ref, hp_ref, x2_ref):
    deg = degp_ref[:, 0] + degp_ref[:, 1] + 1.0
    dinv = lax.rsqrt(deg)
    h = jnp.dot(x_ref[...], wg_ref[...], preferred_element_type=jnp.float32)
    hp_ref[...] = h * dinv[:, None]
    x2_ref[...] = jnp.dot(x_ref[...], wl_ref[...], preferred_element_type=jnp.float32)


def _final_body(accp_ref, hp_ref, x2_ref, degp_ref, b_ref, g_ref, be_ref, out_ref):
    deg = degp_ref[:, 0] + degp_ref[:, 1] + 1.0
    dinv = lax.rsqrt(deg)
    acc = accp_ref[0] + accp_ref[1] + hp_ref[...]
    x1 = dinv[:, None] * acc + b_ref[...]
    z = x1 + x2_ref[...] + 1e-6
    mu = jnp.mean(z, axis=-1, keepdims=True)
    zc = z - mu
    var = jnp.mean(zc * zc, axis=-1, keepdims=True)
    out_ref[...] = zc * lax.rsqrt(var + 1e-5) * g_ref[...] + be_ref[...]


def kernel(adj, x, W_gcn, b_gcn, W_lin, gamma, beta):
    row = adj[0].astype(jnp.int32).reshape(NW * N_LOADS, ROWS_PER_LOAD, 1, CHUNK)
    col = adj[1].astype(jnp.int32).reshape(NW * N_LOADS, ROWS_PER_LOAD, 1, CHUNK)

    zeros_hist = jnp.zeros((PER_TILE,), jnp.float32)
    ones_chunk = jnp.ones((CHUNK,), jnp.float32)
    zeros_rows = jnp.zeros((PER_TILE, D), jnp.float32)

    degp_full = _deg_kernel(col, zeros_hist, ones_chunk)
    degp = degp_full[:, 0, :N].T

    hp, x2 = pl.pallas_call(
        _prep_body,
        grid=(GRID,),
        in_specs=[
            pl.BlockSpec((BLK, D), lambda i: (i, 0)),
            pl.BlockSpec((D, D), lambda i: (0, 0)),
            pl.BlockSpec((D, D), lambda i: (0, 0)),
            pl.BlockSpec((BLK, 2), lambda i: (i, 0)),
        ],
        out_specs=[
            pl.BlockSpec((BLK, D), lambda i: (i, 0)),
            pl.BlockSpec((BLK, D), lambda i: (i, 0)),
        ],
        out_shape=[
            jax.ShapeDtypeStruct((N, D), jnp.float32),
            jax.ShapeDtypeStruct((N, D), jnp.float32),
        ],
    )(x, W_gcn.T, W_lin.T, degp)

    accp_full = _edge_kernel(row, col, hp, zeros_rows)
    accp = accp_full[:, :N, :]

    out = pl.pallas_call(
        _final_body,
        grid=(GRID,),
        in_specs=[
            pl.BlockSpec((2, BLK, D), lambda i: (0, i, 0)),
            pl.BlockSpec((BLK, D), lambda i: (i, 0)),
            pl.BlockSpec((BLK, D), lambda i: (i, 0)),
            pl.BlockSpec((BLK, 2), lambda i: (i, 0)),
            pl.BlockSpec((1, D), lambda i: (0, 0)),
            pl.BlockSpec((1, D), lambda i: (0, 0)),
            pl.BlockSpec((1, D), lambda i: (0, 0)),
        ],
        out_specs=pl.BlockSpec((BLK, D), lambda i: (i, 0)),
        out_shape=jax.ShapeDtypeStruct((N, D), jnp.float32),
    )(accp, hp, x2, degp, b_gcn.reshape(1, D), gamma.reshape(1, D), beta.reshape(1, D))

    return out


# speedup vs baseline: 33.5993x; 1.2987x over previous
import functools

import jax
import jax.numpy as jnp
from jax import lax
from jax.experimental import pallas as pl
from jax.experimental.pallas import tpu as pltpu
from jax.experimental.pallas import tpu_sc as plsc

N = 10000
E = 320000
D = 128

NC = 2
NS = 16
NW = NC * NS

CHUNK = 80
EPW = E // NW
ROWS_PER_LOAD = 25
N_CHUNKS = EPW // CHUNK
N_LOADS = N_CHUNKS // ROWS_PER_LOAD

NPAD = 10240
PER_TILE = NPAD // NS

_mesh = plsc.VectorSubcoreMesh(core_axis_name="c", subcore_axis_name="s")


@functools.partial(
    pl.kernel,
    out_type=jax.ShapeDtypeStruct((NC, 1, NPAD), jnp.float32),
    mesh=_mesh,
    scratch_types=[
        pltpu.VMEM((ROWS_PER_LOAD, 1, CHUNK), jnp.int32),
        pltpu.VMEM((CHUNK,), jnp.float32),
        pltpu.VMEM_SHARED((NPAD,), jnp.float32),
    ],
)
def _deg_kernel(col_hbm, zeros_hbm, ones_hbm, out_hbm, idx_v, ones_v, hist_sp):
    c = lax.axis_index("c")
    s = lax.axis_index("s")
    wid = c * NS + s
    pltpu.sync_copy(zeros_hbm, hist_sp.at[pl.ds(s * PER_TILE, PER_TILE)])
    pltpu.sync_copy(ones_hbm, ones_v)
    plsc.subcore_barrier()

    def outer(o, _):
        pltpu.sync_copy(col_hbm.at[wid, pl.ds(o * ROWS_PER_LOAD, ROWS_PER_LOAD)], idx_v)

        def inner(j, _):
            pltpu.sync_copy(ones_v, hist_sp.at[idx_v.at[j, 0]], add=True)
            return 0

        lax.fori_loop(0, ROWS_PER_LOAD, inner, 0)
        return 0

    lax.fori_loop(0, N_LOADS, outer, 0)
    plsc.subcore_barrier()
    pltpu.sync_copy(
        hist_sp.at[pl.ds(s * PER_TILE, PER_TILE)],
        out_hbm.at[c, 0, pl.ds(s * PER_TILE, PER_TILE)],
    )


N_PAIRS = ROWS_PER_LOAD // 2


@functools.partial(
    pl.kernel,
    out_type=jax.ShapeDtypeStruct((NC, NPAD, D), jnp.float32),
    mesh=_mesh,
    scratch_types=[
        pltpu.VMEM((ROWS_PER_LOAD, 1, CHUNK), jnp.int32),
        pltpu.VMEM((ROWS_PER_LOAD, 1, CHUNK), jnp.int32),
        pltpu.VMEM((ROWS_PER_LOAD, 1, CHUNK), jnp.int32),
        pltpu.VMEM((ROWS_PER_LOAD, 1, CHUNK), jnp.int32),
        pltpu.VMEM((CHUNK, D), jnp.float32),
        pltpu.VMEM((CHUNK, D), jnp.float32),
        pltpu.VMEM_SHARED((NPAD, D), jnp.float32),
        pltpu.SemaphoreType.DMA,
        pltpu.SemaphoreType.DMA,
    ],
)
def _edge_kernel(row_hbm, col_hbm, hp_hbm, zeros_hbm, out_hbm,
                 ir0, ic0, ir1, ic1, b0, b1, acc_sp, sem, isem):
    c = lax.axis_index("c")
    s = lax.axis_index("s")
    wid = c * NS + s
    pltpu.sync_copy(row_hbm.at[wid, pl.ds(0, ROWS_PER_LOAD)], ir0)
    pltpu.sync_copy(col_hbm.at[wid, pl.ds(0, ROWS_PER_LOAD)], ic0)
    pltpu.sync_copy(zeros_hbm, acc_sp.at[pl.ds(s * PER_TILE, PER_TILE)])
    plsc.subcore_barrier()

    def load_block(o, ir, ic):
        base = pl.ds(o * ROWS_PER_LOAD, ROWS_PER_LOAD)
        pltpu.async_copy(row_hbm.at[wid, base], ir, isem)
        pltpu.async_copy(col_hbm.at[wid, base], ic, isem)

    def wait_block(ir, ic):
        pltpu.make_async_copy(row_hbm.at[wid, pl.ds(0, ROWS_PER_LOAD)], ir, isem).wait()
        pltpu.make_async_copy(col_hbm.at[wid, pl.ds(0, ROWS_PER_LOAD)], ic, isem).wait()

    def process_block(ir, ic):
        def grp(g, _):
            d0 = pltpu.async_copy(hp_hbm.at[ir.at[2 * g, 0]], b0, sem)
            d1 = pltpu.async_copy(hp_hbm.at[ir.at[2 * g + 1, 0]], b1, sem)
            d0.wait()
            pltpu.sync_copy(b0, acc_sp.at[ic.at[2 * g, 0]], add=True)
            d1.wait()
            pltpu.sync_copy(b1, acc_sp.at[ic.at[2 * g + 1, 0]], add=True)
            return 0

        lax.fori_loop(0, N_PAIRS, grp, 0)
        j = 2 * N_PAIRS
        pltpu.async_copy(hp_hbm.at[ir.at[j, 0]], b0, sem).wait()
        pltpu.sync_copy(b0, acc_sp.at[ic.at[j, 0]], add=True)

    def outer(o, _):
        even = o % 2 == 0

        @pl.when(jnp.logical_and(even, o < N_LOADS - 1))
        def _():
            load_block(o + 1, ir1, ic1)

        @pl.when(jnp.logical_and(~even, o < N_LOADS - 1))
        def _():
            load_block(o + 1, ir0, ic0)

        @pl.when(even)
        def _():
            process_block(ir0, ic0)

        @pl.when(~even)
        def _():
            process_block(ir1, ic1)

        @pl.when(jnp.logical_and(even, o < N_LOADS - 1))
        def _():
            wait_block(ir1, ic1)

        @pl.when(jnp.logical_and(~even, o < N_LOADS - 1))
        def _():
            wait_block(ir0, ic0)

        return 0

    lax.fori_loop(0, N_LOADS, outer, 0)
    plsc.subcore_barrier()
    pltpu.sync_copy(
        acc_sp.at[pl.ds(s * PER_TILE, PER_TILE)],
        out_hbm.at[c, pl.ds(s * PER_TILE, PER_TILE)],
    )


BLK = 1000
GRID = N // BLK


def _prep_body(x_ref, wg_ref, wl_ref, degp_ref, hp_ref, x2_ref):
    deg = degp_ref[:, 0] + degp_ref[:, 1] + 1.0
    dinv = lax.rsqrt(deg)
    h = jnp.dot(x_ref[...], wg_ref[...], preferred_element_type=jnp.float32)
    hp_ref[...] = h * dinv[:, None]
    x2_ref[...] = jnp.dot(x_ref[...], wl_ref[...], preferred_element_type=jnp.float32)


def _final_body(accp_ref, hp_ref, x2_ref, degp_ref, b_ref, g_ref, be_ref, out_ref):
    deg = degp_ref[:, 0] + degp_ref[:, 1] + 1.0
    dinv = lax.rsqrt(deg)
    acc = accp_ref[0] + accp_ref[1] + hp_ref[...]
    x1 = dinv[:, None] * acc + b_ref[...]
    z = x1 + x2_ref[...] + 1e-6
    mu = jnp.mean(z, axis=-1, keepdims=True)
    zc = z - mu
    var = jnp.mean(zc * zc, axis=-1, keepdims=True)
    out_ref[...] = zc * lax.rsqrt(var + 1e-5) * g_ref[...] + be_ref[...]


def kernel(adj, x, W_gcn, b_gcn, W_lin, gamma, beta):
    row = adj[0].astype(jnp.int32).reshape(NW, N_CHUNKS, 1, CHUNK)
    col = adj[1].astype(jnp.int32).reshape(NW, N_CHUNKS, 1, CHUNK)

    zeros_hist = jnp.zeros((PER_TILE,), jnp.float32)
    ones_chunk = jnp.ones((CHUNK,), jnp.float32)
    zeros_rows = jnp.zeros((PER_TILE, D), jnp.float32)

    degp_full = _deg_kernel(col, zeros_hist, ones_chunk)
    degp = degp_full[:, 0, :N].T

    hp, x2 = pl.pallas_call(
        _prep_body,
        grid=(GRID,),
        in_specs=[
            pl.BlockSpec((BLK, D), lambda i: (i, 0)),
            pl.BlockSpec((D, D), lambda i: (0, 0)),
            pl.BlockSpec((D, D), lambda i: (0, 0)),
            pl.BlockSpec((BLK, 2), lambda i: (i, 0)),
        ],
        out_specs=[
            pl.BlockSpec((BLK, D), lambda i: (i, 0)),
            pl.BlockSpec((BLK, D), lambda i: (i, 0)),
        ],
        out_shape=[
            jax.ShapeDtypeStruct((N, D), jnp.float32),
            jax.ShapeDtypeStruct((N, D), jnp.float32),
        ],
    )(x, W_gcn.T, W_lin.T, degp)

    accp_full = _edge_kernel(row, col, hp, zeros_rows)

    out = pl.pallas_call(
        _final_body,
        grid=(GRID,),
        in_specs=[
            pl.BlockSpec((2, BLK, D), lambda i: (0, i, 0)),
            pl.BlockSpec((BLK, D), lambda i: (i, 0)),
            pl.BlockSpec((BLK, D), lambda i: (i, 0)),
            pl.BlockSpec((BLK, 2), lambda i: (i, 0)),
            pl.BlockSpec((1, D), lambda i: (0, 0)),
            pl.BlockSpec((1, D), lambda i: (0, 0)),
            pl.BlockSpec((1, D), lambda i: (0, 0)),
        ],
        out_specs=pl.BlockSpec((BLK, D), lambda i: (i, 0)),
        out_shape=jax.ShapeDtypeStruct((N, D), jnp.float32),
    )(accp_full, hp, x2, degp, b_gcn.reshape(1, D), gamma.reshape(1, D), beta.reshape(1, D))

    return out


# speedup vs baseline: 37.2686x; 1.1092x over previous
import functools

import jax
import jax.numpy as jnp
from jax import lax
from jax.experimental import pallas as pl
from jax.experimental.pallas import tpu as pltpu
from jax.experimental.pallas import tpu_sc as plsc

N = 10000
E = 320000
D = 128

NC = 2
NS = 16
NW = NC * NS

CHUNK = 80
EPW = E // NW
ROWS_PER_LOAD = 25
N_CHUNKS = EPW // CHUNK
N_LOADS = N_CHUNKS // ROWS_PER_LOAD

NPAD = 10240
PER_TILE = NPAD // NS

_mesh = plsc.VectorSubcoreMesh(core_axis_name="c", subcore_axis_name="s")


@functools.partial(
    pl.kernel,
    out_type=jax.ShapeDtypeStruct((NC, 1, NPAD), jnp.float32),
    mesh=_mesh,
    scratch_types=[
        pltpu.VMEM((ROWS_PER_LOAD, 1, CHUNK), jnp.int32),
        pltpu.VMEM((CHUNK,), jnp.float32),
        pltpu.VMEM_SHARED((NPAD,), jnp.float32),
    ],
)
def _deg_kernel(col_hbm, zeros_hbm, ones_hbm, out_hbm, idx_v, ones_v, hist_sp):
    c = lax.axis_index("c")
    s = lax.axis_index("s")
    wid = c * NS + s
    pltpu.sync_copy(zeros_hbm, hist_sp.at[pl.ds(s * PER_TILE, PER_TILE)])
    pltpu.sync_copy(ones_hbm, ones_v)
    plsc.subcore_barrier()

    def outer(o, _):
        pltpu.sync_copy(col_hbm.at[wid, pl.ds(o * ROWS_PER_LOAD, ROWS_PER_LOAD)], idx_v)

        def inner(j, _):
            pltpu.sync_copy(ones_v, hist_sp.at[idx_v.at[j, 0]], add=True)
            return 0

        lax.fori_loop(0, ROWS_PER_LOAD, inner, 0)
        return 0

    lax.fori_loop(0, N_LOADS, outer, 0)
    plsc.subcore_barrier()
    pltpu.sync_copy(
        hist_sp.at[pl.ds(s * PER_TILE, PER_TILE)],
        out_hbm.at[c, 0, pl.ds(s * PER_TILE, PER_TILE)],
    )


N_PAIRS = ROWS_PER_LOAD // 2


@functools.partial(
    pl.kernel,
    out_type=jax.ShapeDtypeStruct((NC, NPAD, D), jnp.float32),
    mesh=_mesh,
    scratch_types=[
        pltpu.VMEM((ROWS_PER_LOAD, 1, CHUNK), jnp.int32),
        pltpu.VMEM((ROWS_PER_LOAD, 1, CHUNK), jnp.int32),
        pltpu.VMEM((ROWS_PER_LOAD, 1, CHUNK), jnp.int32),
        pltpu.VMEM((ROWS_PER_LOAD, 1, CHUNK), jnp.int32),
        pltpu.VMEM((CHUNK, D), jnp.float32),
        pltpu.VMEM((CHUNK, D), jnp.float32),
        pltpu.VMEM_SHARED((NPAD, D), jnp.float32),
        pltpu.SemaphoreType.DMA,
        pltpu.SemaphoreType.DMA,
        pltpu.SemaphoreType.DMA,
    ],
)
def _edge_kernel(row_hbm, col_hbm, hp_hbm, zeros_hbm, out_hbm,
                 ir0, ic0, ir1, ic1, b0, b1, acc_sp, sem, isem, ssem):
    c = lax.axis_index("c")
    s = lax.axis_index("s")
    wid = c * NS + s
    pltpu.sync_copy(row_hbm.at[wid, pl.ds(0, ROWS_PER_LOAD)], ir0)
    pltpu.sync_copy(col_hbm.at[wid, pl.ds(0, ROWS_PER_LOAD)], ic0)
    pltpu.sync_copy(zeros_hbm, acc_sp.at[pl.ds(s * PER_TILE, PER_TILE)])
    plsc.subcore_barrier()

    def load_block(o, ir, ic):
        base = pl.ds(o * ROWS_PER_LOAD, ROWS_PER_LOAD)
        pltpu.async_copy(row_hbm.at[wid, base], ir, isem)
        pltpu.async_copy(col_hbm.at[wid, base], ic, isem)

    def wait_block(ir, ic):
        pltpu.make_async_copy(row_hbm.at[wid, pl.ds(0, ROWS_PER_LOAD)], ir, isem).wait()
        pltpu.make_async_copy(col_hbm.at[wid, pl.ds(0, ROWS_PER_LOAD)], ic, isem).wait()

    def wait_gather(buf, ir):
        pltpu.make_async_copy(hp_hbm.at[ir.at[0, 0]], buf, sem).wait()

    def wait_scatter(buf, ic):
        pltpu.make_async_copy(buf, acc_sp.at[ic.at[0, 0]], ssem).wait()

    def process_block(ir, ic):
        pltpu.async_copy(hp_hbm.at[ir.at[0, 0]], b0, sem)
        pltpu.async_copy(hp_hbm.at[ir.at[1, 0]], b1, sem)

        def grp(g, _):
            wait_gather(b0, ir)
            pltpu.async_copy(b0, acc_sp.at[ic.at[2 * g, 0]], ssem, add=True)
            wait_gather(b1, ir)
            pltpu.async_copy(b1, acc_sp.at[ic.at[2 * g + 1, 0]], ssem, add=True)
            wait_scatter(b0, ic)
            pltpu.async_copy(hp_hbm.at[ir.at[2 * g + 2, 0]], b0, sem)
            wait_scatter(b1, ic)
            pltpu.async_copy(hp_hbm.at[ir.at[2 * g + 3, 0]], b1, sem)
            return 0

        lax.fori_loop(0, N_PAIRS - 1, grp, 0)
        j = 2 * (N_PAIRS - 1)
        wait_gather(b0, ir)
        pltpu.async_copy(b0, acc_sp.at[ic.at[j, 0]], ssem, add=True)
        wait_gather(b1, ir)
        pltpu.async_copy(b1, acc_sp.at[ic.at[j + 1, 0]], ssem, add=True)
        wait_scatter(b0, ic)
        pltpu.async_copy(hp_hbm.at[ir.at[j + 2, 0]], b0, sem)
        wait_scatter(b1, ic)
        wait_gather(b0, ir)
        pltpu.sync_copy(b0, acc_sp.at[ic.at[j + 2, 0]], add=True)

    def outer(o, _):
        even = o % 2 == 0

        @pl.when(jnp.logical_and(even, o < N_LOADS - 1))
        def _():
            load_block(o + 1, ir1, ic1)

        @pl.when(jnp.logical_and(~even, o < N_LOADS - 1))
        def _():
            load_block(o + 1, ir0, ic0)

        @pl.when(even)
        def _():
            process_block(ir0, ic0)

        @pl.when(~even)
        def _():
            process_block(ir1, ic1)

        @pl.when(jnp.logical_and(even, o < N_LOADS - 1))
        def _():
            wait_block(ir1, ic1)

        @pl.when(jnp.logical_and(~even, o < N_LOADS - 1))
        def _():
            wait_block(ir0, ic0)

        return 0

    lax.fori_loop(0, N_LOADS, outer, 0)
    plsc.subcore_barrier()
    pltpu.sync_copy(
        acc_sp.at[pl.ds(s * PER_TILE, PER_TILE)],
        out_hbm.at[c, pl.ds(s * PER_TILE, PER_TILE)],
    )


BLK = 1000
GRID = N // BLK


def _prep_body(x_ref, wg_ref, wl_ref, degp_ref, hp_ref, x2_ref):
    deg = degp_ref[:, 0] + degp_ref[:, 1] + 1.0
    dinv = lax.rsqrt(deg)
    h = jnp.dot(x_ref[...], wg_ref[...], preferred_element_type=jnp.float32)
    hp_ref[...] = h * dinv[:, None]
    x2_ref[...] = jnp.dot(x_ref[...], wl_ref[...], preferred_element_type=jnp.float32)


def _final_body(accp_ref, hp_ref, x2_ref, degp_ref, b_ref, g_ref, be_ref, out_ref):
    deg = degp_ref[:, 0] + degp_ref[:, 1] + 1.0
    dinv = lax.rsqrt(deg)
    acc = accp_ref[0] + accp_ref[1] + hp_ref[...]
    x1 = dinv[:, None] * acc + b_ref[...]
    z = x1 + x2_ref[...] + 1e-6
    mu = jnp.mean(z, axis=-1, keepdims=True)
    zc = z - mu
    var = jnp.mean(zc * zc, axis=-1, keepdims=True)
    out_ref[...] = zc * lax.rsqrt(var + 1e-5) * g_ref[...] + be_ref[...]


def kernel(adj, x, W_gcn, b_gcn, W_lin, gamma, beta):
    row = adj[0].astype(jnp.int32).reshape(NW, N_CHUNKS, 1, CHUNK)
    col = adj[1].astype(jnp.int32).reshape(NW, N_CHUNKS, 1, CHUNK)

    zeros_hist = jnp.zeros((PER_TILE,), jnp.float32)
    ones_chunk = jnp.ones((CHUNK,), jnp.float32)
    zeros_rows = jnp.zeros((PER_TILE, D), jnp.float32)

    degp_full = _deg_kernel(col, zeros_hist, ones_chunk)
    degp = degp_full[:, 0, :N].T

    hp, x2 = pl.pallas_call(
        _prep_body,
        grid=(GRID,),
        in_specs=[
            pl.BlockSpec((BLK, D), lambda i: (i, 0)),
            pl.BlockSpec((D, D), lambda i: (0, 0)),
            pl.BlockSpec((D, D), lambda i: (0, 0)),
            pl.BlockSpec((BLK, 2), lambda i: (i, 0)),
        ],
        out_specs=[
            pl.BlockSpec((BLK, D), lambda i: (i, 0)),
            pl.BlockSpec((BLK, D), lambda i: (i, 0)),
        ],
        out_shape=[
            jax.ShapeDtypeStruct((N, D), jnp.float32),
            jax.ShapeDtypeStruct((N, D), jnp.float32),
        ],
    )(x, W_gcn.T, W_lin.T, degp)

    accp_full = _edge_kernel(row, col, hp, zeros_rows)

    out = pl.pallas_call(
        _final_body,
        grid=(GRID,),
        in_specs=[
            pl.BlockSpec((2, BLK, D), lambda i: (0, i, 0)),
            pl.BlockSpec((BLK, D), lambda i: (i, 0)),
            pl.BlockSpec((BLK, D), lambda i: (i, 0)),
            pl.BlockSpec((BLK, 2), lambda i: (i, 0)),
            pl.BlockSpec((1, D), lambda i: (0, 0)),
            pl.BlockSpec((1, D), lambda i: (0, 0)),
            pl.BlockSpec((1, D), lambda i: (0, 0)),
        ],
        out_specs=pl.BlockSpec((BLK, D), lambda i: (i, 0)),
        out_shape=jax.ShapeDtypeStruct((N, D), jnp.float32),
    )(accp_full, hp, x2, degp, b_gcn.reshape(1, D), gamma.reshape(1, D), beta.reshape(1, D))

    return out


# speedup vs baseline: 38.8264x; 1.0418x over previous
import functools

import jax
import jax.numpy as jnp
from jax import lax
from jax.experimental import pallas as pl
from jax.experimental.pallas import tpu as pltpu
from jax.experimental.pallas import tpu_sc as plsc

N = 10000
E = 320000
D = 128

NC = 2
NS = 16
NW = NC * NS

CHUNK = 128
EPW = 10240
EPAD = NW * EPW
ROWS_PER_LOAD = 10
N_CHUNKS = EPW // CHUNK
N_LOADS = N_CHUNKS // ROWS_PER_LOAD
DEG_ROWS_PER_LOAD = 5
DEG_N_LOADS = N_CHUNKS // DEG_ROWS_PER_LOAD

NPAD = 10240
PER_TILE = NPAD // NS

_mesh = plsc.VectorSubcoreMesh(core_axis_name="c", subcore_axis_name="s")


@functools.partial(
    pl.kernel,
    out_type=jax.ShapeDtypeStruct((NC, 1, NPAD), jnp.float32),
    mesh=_mesh,
    scratch_types=[
        pltpu.VMEM((DEG_ROWS_PER_LOAD, 1, CHUNK), jnp.int32),
        pltpu.VMEM((CHUNK,), jnp.float32),
        pltpu.VMEM_SHARED((NPAD,), jnp.float32),
    ],
)
def _deg_kernel(col_hbm, zeros_hbm, ones_hbm, out_hbm, idx_v, ones_v, hist_sp):
    c = lax.axis_index("c")
    s = lax.axis_index("s")
    wid = c * NS + s
    pltpu.sync_copy(zeros_hbm, hist_sp.at[pl.ds(s * PER_TILE, PER_TILE)])
    pltpu.sync_copy(ones_hbm, ones_v)
    plsc.subcore_barrier()

    def outer(o, _):
        pltpu.sync_copy(
            col_hbm.at[wid, pl.ds(o * DEG_ROWS_PER_LOAD, DEG_ROWS_PER_LOAD)], idx_v
        )

        def inner(j, _):
            pltpu.sync_copy(ones_v, hist_sp.at[idx_v.at[j, 0]], add=True)
            return 0

        lax.fori_loop(0, DEG_ROWS_PER_LOAD, inner, 0)
        return 0

    lax.fori_loop(0, DEG_N_LOADS, outer, 0)
    plsc.subcore_barrier()
    pltpu.sync_copy(
        hist_sp.at[pl.ds(s * PER_TILE, PER_TILE)],
        out_hbm.at[c, 0, pl.ds(s * PER_TILE, PER_TILE)],
    )


N_PAIRS = ROWS_PER_LOAD // 2


@functools.partial(
    pl.kernel,
    out_type=jax.ShapeDtypeStruct((NC, NPAD, D), jnp.float32),
    mesh=_mesh,
    scratch_types=[
        pltpu.VMEM((ROWS_PER_LOAD, 1, CHUNK), jnp.int32),
        pltpu.VMEM((ROWS_PER_LOAD, 1, CHUNK), jnp.int32),
        pltpu.VMEM((ROWS_PER_LOAD, 1, CHUNK), jnp.int32),
        pltpu.VMEM((ROWS_PER_LOAD, 1, CHUNK), jnp.int32),
        pltpu.VMEM((CHUNK, D), jnp.float32),
        pltpu.VMEM((CHUNK, D), jnp.float32),
        pltpu.VMEM_SHARED((NPAD, D), jnp.float32),
        pltpu.SemaphoreType.DMA,
        pltpu.SemaphoreType.DMA,
        pltpu.SemaphoreType.DMA,
    ],
)
def _edge_kernel(row_hbm, col_hbm, hp_hbm, zeros_hbm, out_hbm,
                 ir0, ic0, ir1, ic1, b0, b1, acc_sp, sem, isem, ssem):
    c = lax.axis_index("c")
    s = lax.axis_index("s")
    wid = c * NS + s
    pltpu.sync_copy(row_hbm.at[wid, pl.ds(0, ROWS_PER_LOAD)], ir0)
    pltpu.sync_copy(col_hbm.at[wid, pl.ds(0, ROWS_PER_LOAD)], ic0)
    pltpu.sync_copy(zeros_hbm, acc_sp.at[pl.ds(s * PER_TILE, PER_TILE)])
    plsc.subcore_barrier()

    def load_block(o, ir, ic):
        base = pl.ds(o * ROWS_PER_LOAD, ROWS_PER_LOAD)
        pltpu.async_copy(row_hbm.at[wid, base], ir, isem)
        pltpu.async_copy(col_hbm.at[wid, base], ic, isem)

    def wait_block(ir, ic):
        pltpu.make_async_copy(row_hbm.at[wid, pl.ds(0, ROWS_PER_LOAD)], ir, isem).wait()
        pltpu.make_async_copy(col_hbm.at[wid, pl.ds(0, ROWS_PER_LOAD)], ic, isem).wait()

    def wait_gather(buf, ir):
        pltpu.make_async_copy(hp_hbm.at[ir.at[0, 0]], buf, sem).wait()

    def wait_scatter(buf, ic):
        pltpu.make_async_copy(buf, acc_sp.at[ic.at[0, 0]], ssem).wait()

    def process_block(ir, ic):
        pltpu.async_copy(hp_hbm.at[ir.at[0, 0]], b0, sem)
        pltpu.async_copy(hp_hbm.at[ir.at[1, 0]], b1, sem)

        def grp(g, _):
            wait_gather(b0, ir)
            pltpu.async_copy(b0, acc_sp.at[ic.at[2 * g, 0]], ssem, add=True)
            wait_gather(b1, ir)
            pltpu.async_copy(b1, acc_sp.at[ic.at[2 * g + 1, 0]], ssem, add=True)
            wait_scatter(b0, ic)
            pltpu.async_copy(hp_hbm.at[ir.at[2 * g + 2, 0]], b0, sem)
            wait_scatter(b1, ic)
            pltpu.async_copy(hp_hbm.at[ir.at[2 * g + 3, 0]], b1, sem)
            return 0

        lax.fori_loop(0, N_PAIRS - 1, grp, 0)
        j = 2 * (N_PAIRS - 1)
        wait_gather(b0, ir)
        pltpu.async_copy(b0, acc_sp.at[ic.at[j, 0]], ssem, add=True)
        wait_gather(b1, ir)
        pltpu.async_copy(b1, acc_sp.at[ic.at[j + 1, 0]], ssem, add=True)
        wait_scatter(b0, ic)
        wait_scatter(b1, ic)

    def outer(o, _):
        even = o % 2 == 0

        @pl.when(jnp.logical_and(even, o < N_LOADS - 1))
        def _():
            load_block(o + 1, ir1, ic1)

        @pl.when(jnp.logical_and(~even, o < N_LOADS - 1))
        def _():
            load_block(o + 1, ir0, ic0)

        @pl.when(even)
        def _():
            process_block(ir0, ic0)

        @pl.when(~even)
        def _():
            process_block(ir1, ic1)

        @pl.when(jnp.logical_and(even, o < N_LOADS - 1))
        def _():
            wait_block(ir1, ic1)

        @pl.when(jnp.logical_and(~even, o < N_LOADS - 1))
        def _():
            wait_block(ir0, ic0)

        return 0

    lax.fori_loop(0, N_LOADS, outer, 0)
    plsc.subcore_barrier()
    pltpu.sync_copy(
        acc_sp.at[pl.ds(s * PER_TILE, PER_TILE)],
        out_hbm.at[c, pl.ds(s * PER_TILE, PER_TILE)],
    )


BLK = 1000
GRID = N // BLK


def _prep_body(x_ref, wg_ref, wl_ref, degp_ref, hp_ref, x2_ref):
    deg = degp_ref[:, 0] + degp_ref[:, 1] + 1.0
    dinv = lax.rsqrt(deg)
    h = jnp.dot(x_ref[...], wg_ref[...], preferred_element_type=jnp.float32)
    hp_ref[...] = h * dinv[:, None]
    x2_ref[...] = jnp.dot(x_ref[...], wl_ref[...], preferred_element_type=jnp.float32)


def _final_body(accp_ref, hp_ref, x2_ref, degp_ref, b_ref, g_ref, be_ref, out_ref):
    deg = degp_ref[:, 0] + degp_ref[:, 1] + 1.0
    dinv = lax.rsqrt(deg)
    acc = accp_ref[0] + accp_ref[1] + hp_ref[...]
    x1 = dinv[:, None] * acc + b_ref[...]
    z = x1 + x2_ref[...] + 1e-6
    mu = jnp.mean(z, axis=-1, keepdims=True)
    zc = z - mu
    var = jnp.mean(zc * zc, axis=-1, keepdims=True)
    out_ref[...] = zc * lax.rsqrt(var + 1e-5) * g_ref[...] + be_ref[...]


def kernel(adj, x, W_gcn, b_gcn, W_lin, gamma, beta):
    n_extra = EPAD - E
    pad_row = jnp.arange(n_extra, dtype=jnp.int32) % N
    pad_col = N + jnp.arange(n_extra, dtype=jnp.int32) % (NPAD - N)
    row = jnp.concatenate([adj[0].astype(jnp.int32), pad_row])
    col = jnp.concatenate([adj[1].astype(jnp.int32), pad_col])
    row = row.reshape(NW, N_CHUNKS, 1, CHUNK)
    col = col.reshape(NW, N_CHUNKS, 1, CHUNK)

    zeros_hist = jnp.zeros((PER_TILE,), jnp.float32)
    ones_chunk = jnp.ones((CHUNK,), jnp.float32)
    zeros_rows = jnp.zeros((PER_TILE, D), jnp.float32)

    degp_full = _deg_kernel(col, zeros_hist, ones_chunk)
    degp = degp_full[:, 0, :N].T

    hp, x2 = pl.pallas_call(
        _prep_body,
        grid=(GRID,),
        in_specs=[
            pl.BlockSpec((BLK, D), lambda i: (i, 0)),
            pl.BlockSpec((D, D), lambda i: (0, 0)),
            pl.BlockSpec((D, D), lambda i: (0, 0)),
            pl.BlockSpec((BLK, 2), lambda i: (i, 0)),
        ],
        out_specs=[
            pl.BlockSpec((BLK, D), lambda i: (i, 0)),
            pl.BlockSpec((BLK, D), lambda i: (i, 0)),
        ],
        out_shape=[
            jax.ShapeDtypeStruct((N, D), jnp.float32),
            jax.ShapeDtypeStruct((N, D), jnp.float32),
        ],
    )(x, W_gcn.T, W_lin.T, degp)

    accp_full = _edge_kernel(row, col, hp, zeros_rows)

    out = pl.pallas_call(
        _final_body,
        grid=(GRID,),
        in_specs=[
            pl.BlockSpec((2, BLK, D), lambda i: (0, i, 0)),
            pl.BlockSpec((BLK, D), lambda i: (i, 0)),
            pl.BlockSpec((BLK, D), lambda i: (i, 0)),
            pl.BlockSpec((BLK, 2), lambda i: (i, 0)),
            pl.BlockSpec((1, D), lambda i: (0, 0)),
            pl.BlockSpec((1, D), lambda i: (0, 0)),
            pl.BlockSpec((1, D), lambda i: (0, 0)),
        ],
        out_specs=pl.BlockSpec((BLK, D), lambda i: (i, 0)),
        out_shape=jax.ShapeDtypeStruct((N, D), jnp.float32),
    )(accp_full, hp, x2, degp, b_gcn.reshape(1, D), gamma.reshape(1, D), beta.reshape(1, D))

    return out


# speedup vs baseline: 39.0964x; 1.0070x over previous
import functools

import jax
import jax.numpy as jnp
from jax import lax
from jax.experimental import pallas as pl
from jax.experimental.pallas import tpu as pltpu
from jax.experimental.pallas import tpu_sc as plsc

N = 10000
E = 320000
D = 128

NC = 2
NS = 16
NW = NC * NS

CHUNK = 128
EPW = 10240
EPAD = NW * EPW
ROWS_PER_LOAD = 10
N_CHUNKS = EPW // CHUNK
N_LOADS = N_CHUNKS // ROWS_PER_LOAD
DEG_ROWS_PER_LOAD = 5
DEG_N_LOADS = N_CHUNKS // DEG_ROWS_PER_LOAD

NPAD = 10240
PER_TILE = NPAD // NS

_mesh = plsc.VectorSubcoreMesh(core_axis_name="c", subcore_axis_name="s")


@functools.partial(
    pl.kernel,
    out_type=jax.ShapeDtypeStruct((NC, 1, NPAD), jnp.float32),
    mesh=_mesh,
    scratch_types=[
        pltpu.VMEM((DEG_ROWS_PER_LOAD, 1, CHUNK), jnp.int32),
        pltpu.VMEM((CHUNK,), jnp.float32),
        pltpu.VMEM_SHARED((NPAD,), jnp.float32),
    ],
)
def _deg_kernel(col_hbm, zeros_hbm, ones_hbm, out_hbm, idx_v, ones_v, hist_sp):
    c = lax.axis_index("c")
    s = lax.axis_index("s")
    wid = c * NS + s
    pltpu.sync_copy(zeros_hbm, hist_sp.at[pl.ds(s * PER_TILE, PER_TILE)])
    pltpu.sync_copy(ones_hbm, ones_v)
    plsc.subcore_barrier()

    def outer(o, _):
        pltpu.sync_copy(
            col_hbm.at[wid, pl.ds(o * DEG_ROWS_PER_LOAD, DEG_ROWS_PER_LOAD)], idx_v
        )

        def inner(j, _):
            pltpu.sync_copy(ones_v, hist_sp.at[idx_v.at[j, 0]], add=True)
            return 0

        lax.fori_loop(0, DEG_ROWS_PER_LOAD, inner, 0)
        return 0

    lax.fori_loop(0, DEG_N_LOADS, outer, 0)
    plsc.subcore_barrier()
    pltpu.sync_copy(
        hist_sp.at[pl.ds(s * PER_TILE, PER_TILE)],
        out_hbm.at[c, 0, pl.ds(s * PER_TILE, PER_TILE)],
    )


N_PAIRS = ROWS_PER_LOAD // 2


@functools.partial(
    pl.kernel,
    out_type=jax.ShapeDtypeStruct((NC, NPAD, D), jnp.float32),
    mesh=_mesh,
    scratch_types=[
        pltpu.VMEM((ROWS_PER_LOAD, 1, CHUNK), jnp.int32),
        pltpu.VMEM((ROWS_PER_LOAD, 1, CHUNK), jnp.int32),
        pltpu.VMEM((ROWS_PER_LOAD, 1, CHUNK), jnp.int32),
        pltpu.VMEM((ROWS_PER_LOAD, 1, CHUNK), jnp.int32),
        pltpu.VMEM((CHUNK, D), jnp.float32),
        pltpu.VMEM((CHUNK, D), jnp.float32),
        pltpu.VMEM_SHARED((NPAD, D), jnp.float32),
        pltpu.SemaphoreType.DMA,
        pltpu.SemaphoreType.DMA,
        pltpu.SemaphoreType.DMA,
    ],
)
def _edge_kernel(row_hbm, col_hbm, hp_hbm, zeros_hbm, out_hbm,
                 ir0, ic0, ir1, ic1, b0, b1, acc_sp, sem, isem, ssem):
    c = lax.axis_index("c")
    s = lax.axis_index("s")
    wid = c * NS + s
    pltpu.sync_copy(row_hbm.at[wid, pl.ds(0, ROWS_PER_LOAD)], ir0)
    pltpu.sync_copy(col_hbm.at[wid, pl.ds(0, ROWS_PER_LOAD)], ic0)
    pltpu.sync_copy(zeros_hbm, acc_sp.at[pl.ds(s * PER_TILE, PER_TILE)])
    plsc.subcore_barrier()

    def load_block(o, ir, ic):
        base = pl.ds(o * ROWS_PER_LOAD, ROWS_PER_LOAD)
        pltpu.async_copy(row_hbm.at[wid, base], ir, isem)
        pltpu.async_copy(col_hbm.at[wid, base], ic, isem)

    def wait_block(ir, ic):
        pltpu.make_async_copy(row_hbm.at[wid, pl.ds(0, ROWS_PER_LOAD)], ir, isem).wait()
        pltpu.make_async_copy(col_hbm.at[wid, pl.ds(0, ROWS_PER_LOAD)], ic, isem).wait()

    def wait_gather(buf, ir):
        pltpu.make_async_copy(hp_hbm.at[ir.at[0, 0]], buf, sem).wait()

    def wait_scatter(buf, ic):
        pltpu.make_async_copy(buf, acc_sp.at[ic.at[0, 0]], ssem).wait()

    def process_block(ir, ic):
        pltpu.async_copy(hp_hbm.at[ir.at[0, 0]], b0, sem)
        pltpu.async_copy(hp_hbm.at[ir.at[1, 0]], b1, sem)

        def grp(g, _):
            wait_gather(b0, ir)
            pltpu.async_copy(b0, acc_sp.at[ic.at[2 * g, 0]], ssem, add=True)
            wait_gather(b1, ir)
            pltpu.async_copy(b1, acc_sp.at[ic.at[2 * g + 1, 0]], ssem, add=True)
            wait_scatter(b0, ic)
            pltpu.async_copy(hp_hbm.at[ir.at[2 * g + 2, 0]], b0, sem)
            wait_scatter(b1, ic)
            pltpu.async_copy(hp_hbm.at[ir.at[2 * g + 3, 0]], b1, sem)
            return 0

        lax.fori_loop(0, N_PAIRS - 1, grp, 0)
        j = 2 * (N_PAIRS - 1)
        wait_gather(b0, ir)
        pltpu.async_copy(b0, acc_sp.at[ic.at[j, 0]], ssem, add=True)
        wait_gather(b1, ir)
        pltpu.async_copy(b1, acc_sp.at[ic.at[j + 1, 0]], ssem, add=True)
        wait_scatter(b0, ic)
        wait_scatter(b1, ic)

    def outer(o, _):
        even = o % 2 == 0

        @pl.when(jnp.logical_and(even, o < N_LOADS - 1))
        def _():
            load_block(o + 1, ir1, ic1)

        @pl.when(jnp.logical_and(~even, o < N_LOADS - 1))
        def _():
            load_block(o + 1, ir0, ic0)

        @pl.when(even)
        def _():
            process_block(ir0, ic0)

        @pl.when(~even)
        def _():
            process_block(ir1, ic1)

        @pl.when(jnp.logical_and(even, o < N_LOADS - 1))
        def _():
            wait_block(ir1, ic1)

        @pl.when(jnp.logical_and(~even, o < N_LOADS - 1))
        def _():
            wait_block(ir0, ic0)

        return 0

    lax.fori_loop(0, N_LOADS, outer, 0)
    plsc.subcore_barrier()
    pltpu.sync_copy(
        acc_sp.at[pl.ds(s * PER_TILE, PER_TILE)],
        out_hbm.at[c, pl.ds(s * PER_TILE, PER_TILE)],
    )


BLK = 1000
GRID = N // BLK


def _mm_body(x_ref, wg_ref, wl_ref, h_ref, x2_ref):
    h_ref[...] = jnp.dot(x_ref[...], wg_ref[...], preferred_element_type=jnp.float32)
    x2_ref[...] = jnp.dot(x_ref[...], wl_ref[...], preferred_element_type=jnp.float32)


def _scale_body(h_ref, degp_ref, hp_ref):
    deg = degp_ref[:, 0] + degp_ref[:, 1] + 1.0
    hp_ref[...] = h_ref[...] * lax.rsqrt(deg)[:, None]


def _final_body(accp_ref, hp_ref, x2_ref, degp_ref, b_ref, g_ref, be_ref, out_ref):
    deg = degp_ref[:, 0] + degp_ref[:, 1] + 1.0
    dinv = lax.rsqrt(deg)
    acc = accp_ref[0] + accp_ref[1] + hp_ref[...]
    x1 = dinv[:, None] * acc + b_ref[...]
    z = x1 + x2_ref[...] + 1e-6
    mu = jnp.mean(z, axis=-1, keepdims=True)
    zc = z - mu
    var = jnp.mean(zc * zc, axis=-1, keepdims=True)
    out_ref[...] = zc * lax.rsqrt(var + 1e-5) * g_ref[...] + be_ref[...]


def kernel(adj, x, W_gcn, b_gcn, W_lin, gamma, beta):
    n_extra = EPAD - E
    pad_row = jnp.arange(n_extra, dtype=jnp.int32) % N
    pad_col = N + jnp.arange(n_extra, dtype=jnp.int32) % (NPAD - N)
    row = jnp.concatenate([adj[0].astype(jnp.int32), pad_row])
    col = jnp.concatenate([adj[1].astype(jnp.int32), pad_col])
    row = row.reshape(NW, N_CHUNKS, 1, CHUNK)
    col = col.reshape(NW, N_CHUNKS, 1, CHUNK)

    zeros_hist = jnp.zeros((PER_TILE,), jnp.float32)
    ones_chunk = jnp.ones((CHUNK,), jnp.float32)
    zeros_rows = jnp.zeros((PER_TILE, D), jnp.float32)

    degp_full = _deg_kernel(col, zeros_hist, ones_chunk)
    degp = degp_full[:, 0, :N].T

    h_raw, x2 = pl.pallas_call(
        _mm_body,
        grid=(GRID,),
        in_specs=[
            pl.BlockSpec((BLK, D), lambda i: (i, 0)),
            pl.BlockSpec((D, D), lambda i: (0, 0)),
            pl.BlockSpec((D, D), lambda i: (0, 0)),
        ],
        out_specs=[
            pl.BlockSpec((BLK, D), lambda i: (i, 0)),
            pl.BlockSpec((BLK, D), lambda i: (i, 0)),
        ],
        out_shape=[
            jax.ShapeDtypeStruct((N, D), jnp.float32),
            jax.ShapeDtypeStruct((N, D), jnp.float32),
        ],
    )(x, W_gcn.T, W_lin.T)

    hp = pl.pallas_call(
        _scale_body,
        grid=(GRID,),
        in_specs=[
            pl.BlockSpec((BLK, D), lambda i: (i, 0)),
            pl.BlockSpec((BLK, 2), lambda i: (i, 0)),
        ],
        out_specs=pl.BlockSpec((BLK, D), lambda i: (i, 0)),
        out_shape=jax.ShapeDtypeStruct((N, D), jnp.float32),
    )(h_raw, degp)

    accp_full = _edge_kernel(row, col, hp, zeros_rows)

    out = pl.pallas_call(
        _final_body,
        grid=(GRID,),
        in_specs=[
            pl.BlockSpec((2, BLK, D), lambda i: (0, i, 0)),
            pl.BlockSpec((BLK, D), lambda i: (i, 0)),
            pl.BlockSpec((BLK, D), lambda i: (i, 0)),
            pl.BlockSpec((BLK, 2), lambda i: (i, 0)),
            pl.BlockSpec((1, D), lambda i: (0, 0)),
            pl.BlockSpec((1, D), lambda i: (0, 0)),
            pl.BlockSpec((1, D), lambda i: (0, 0)),
        ],
        out_specs=pl.BlockSpec((BLK, D), lambda i: (i, 0)),
        out_shape=jax.ShapeDtypeStruct((N, D), jnp.float32),
    )(accp_full, hp, x2, degp, b_gcn.reshape(1, D), gamma.reshape(1, D), beta.reshape(1, D))

    return out


# speedup vs baseline: 39.7523x; 1.0168x over previous
import functools

import jax
import jax.numpy as jnp
from jax import lax
from jax.experimental import pallas as pl
from jax.experimental.pallas import tpu as pltpu
from jax.experimental.pallas import tpu_sc as plsc

N = 10000
E = 320000
D = 128

NC = 2
NS = 16
NW = NC * NS

CHUNK = 128
EPW = 10240
EPAD = NW * EPW
ROWS_PER_LOAD = 10
N_CHUNKS = EPW // CHUNK
N_LOADS = N_CHUNKS // ROWS_PER_LOAD
DEG_ROWS_PER_LOAD = 5
DEG_N_LOADS = N_CHUNKS // DEG_ROWS_PER_LOAD

NPAD = 10240
PER_TILE = NPAD // NS

_mesh = plsc.VectorSubcoreMesh(core_axis_name="c", subcore_axis_name="s")


@functools.partial(
    pl.kernel,
    out_type=jax.ShapeDtypeStruct((NC, 1, NPAD), jnp.float32),
    mesh=_mesh,
    scratch_types=[
        pltpu.VMEM((DEG_ROWS_PER_LOAD, 1, CHUNK), jnp.int32),
        pltpu.VMEM((CHUNK,), jnp.float32),
        pltpu.VMEM_SHARED((NPAD,), jnp.float32),
        pltpu.SemaphoreType.DMA,
    ],
)
def _deg_kernel(col_hbm, zeros_hbm, ones_hbm, out_hbm, idx_v, ones_v, hist_sp, dsem):
    c = lax.axis_index("c")
    s = lax.axis_index("s")
    wid = c * NS + s
    pltpu.sync_copy(zeros_hbm, hist_sp.at[pl.ds(s * PER_TILE, PER_TILE)])
    pltpu.sync_copy(ones_hbm, ones_v)
    plsc.subcore_barrier()

    def outer(o, _):
        pltpu.sync_copy(
            col_hbm.at[wid, pl.ds(o * DEG_ROWS_PER_LOAD, DEG_ROWS_PER_LOAD)], idx_v
        )
        for j in range(DEG_ROWS_PER_LOAD):
            pltpu.async_copy(ones_v, hist_sp.at[idx_v.at[j, 0]], dsem, add=True)
        for j in range(DEG_ROWS_PER_LOAD):
            pltpu.make_async_copy(ones_v, hist_sp.at[idx_v.at[0, 0]], dsem).wait()
        return 0

    lax.fori_loop(0, DEG_N_LOADS, outer, 0)
    plsc.subcore_barrier()
    pltpu.sync_copy(
        hist_sp.at[pl.ds(s * PER_TILE, PER_TILE)],
        out_hbm.at[c, 0, pl.ds(s * PER_TILE, PER_TILE)],
    )


N_PAIRS = ROWS_PER_LOAD // 2


@functools.partial(
    pl.kernel,
    out_type=jax.ShapeDtypeStruct((NC, NPAD, D), jnp.float32),
    mesh=_mesh,
    scratch_types=[
        pltpu.VMEM((ROWS_PER_LOAD, 1, CHUNK), jnp.int32),
        pltpu.VMEM((ROWS_PER_LOAD, 1, CHUNK), jnp.int32),
        pltpu.VMEM((ROWS_PER_LOAD, 1, CHUNK), jnp.int32),
        pltpu.VMEM((ROWS_PER_LOAD, 1, CHUNK), jnp.int32),
        pltpu.VMEM((CHUNK, D), jnp.float32),
        pltpu.VMEM((CHUNK, D), jnp.float32),
        pltpu.VMEM_SHARED((NPAD, D), jnp.float32),
        pltpu.SemaphoreType.DMA,
        pltpu.SemaphoreType.DMA,
        pltpu.SemaphoreType.DMA,
    ],
)
def _edge_kernel(row_hbm, col_hbm, hp_hbm, zeros_hbm, out_hbm,
                 ir0, ic0, ir1, ic1, b0, b1, acc_sp, sem, isem, ssem):
    c = lax.axis_index("c")
    s = lax.axis_index("s")
    wid = c * NS + s
    pltpu.sync_copy(row_hbm.at[wid, pl.ds(0, ROWS_PER_LOAD)], ir0)
    pltpu.sync_copy(col_hbm.at[wid, pl.ds(0, ROWS_PER_LOAD)], ic0)
    pltpu.sync_copy(zeros_hbm, acc_sp.at[pl.ds(s * PER_TILE, PER_TILE)])
    plsc.subcore_barrier()

    def load_block(o, ir, ic):
        base = pl.ds(o * ROWS_PER_LOAD, ROWS_PER_LOAD)
        pltpu.async_copy(row_hbm.at[wid, base], ir, isem)
        pltpu.async_copy(col_hbm.at[wid, base], ic, isem)

    def wait_block(ir, ic):
        pltpu.make_async_copy(row_hbm.at[wid, pl.ds(0, ROWS_PER_LOAD)], ir, isem).wait()
        pltpu.make_async_copy(col_hbm.at[wid, pl.ds(0, ROWS_PER_LOAD)], ic, isem).wait()

    def wait_gather(buf, ir):
        pltpu.make_async_copy(hp_hbm.at[ir.at[0, 0]], buf, sem).wait()

    def wait_scatter(buf, ic):
        pltpu.make_async_copy(buf, acc_sp.at[ic.at[0, 0]], ssem).wait()

    def process_block(ir, ic):
        pltpu.async_copy(hp_hbm.at[ir.at[0, 0]], b0, sem)
        pltpu.async_copy(hp_hbm.at[ir.at[1, 0]], b1, sem)

        def grp(g, _):
            wait_gather(b0, ir)
            pltpu.async_copy(b0, acc_sp.at[ic.at[2 * g, 0]], ssem, add=True)
            wait_gather(b1, ir)
            pltpu.async_copy(b1, acc_sp.at[ic.at[2 * g + 1, 0]], ssem, add=True)
            wait_scatter(b0, ic)
            pltpu.async_copy(hp_hbm.at[ir.at[2 * g + 2, 0]], b0, sem)
            wait_scatter(b1, ic)
            pltpu.async_copy(hp_hbm.at[ir.at[2 * g + 3, 0]], b1, sem)
            return 0

        lax.fori_loop(0, N_PAIRS - 1, grp, 0)
        j = 2 * (N_PAIRS - 1)
        wait_gather(b0, ir)
        pltpu.async_copy(b0, acc_sp.at[ic.at[j, 0]], ssem, add=True)
        wait_gather(b1, ir)
        pltpu.async_copy(b1, acc_sp.at[ic.at[j + 1, 0]], ssem, add=True)
        wait_scatter(b0, ic)
        wait_scatter(b1, ic)

    def outer(o, _):
        even = o % 2 == 0

        @pl.when(jnp.logical_and(even, o < N_LOADS - 1))
        def _():
            load_block(o + 1, ir1, ic1)

        @pl.when(jnp.logical_and(~even, o < N_LOADS - 1))
        def _():
            load_block(o + 1, ir0, ic0)

        @pl.when(even)
        def _():
            process_block(ir0, ic0)

        @pl.when(~even)
        def _():
            process_block(ir1, ic1)

        @pl.when(jnp.logical_and(even, o < N_LOADS - 1))
        def _():
            wait_block(ir1, ic1)

        @pl.when(jnp.logical_and(~even, o < N_LOADS - 1))
        def _():
            wait_block(ir0, ic0)

        return 0

    lax.fori_loop(0, N_LOADS, outer, 0)
    plsc.subcore_barrier()
    pltpu.sync_copy(
        acc_sp.at[pl.ds(s * PER_TILE, PER_TILE)],
        out_hbm.at[c, pl.ds(s * PER_TILE, PER_TILE)],
    )


BLK = 1000
GRID = N // BLK


def _mm_body(x_ref, wg_ref, wl_ref, h_ref, x2_ref):
    h_ref[...] = jnp.dot(x_ref[...], wg_ref[...], preferred_element_type=jnp.float32)
    x2_ref[...] = jnp.dot(x_ref[...], wl_ref[...], preferred_element_type=jnp.float32)


def _scale_body(h_ref, degp_ref, hp_ref):
    deg = degp_ref[:, 0] + degp_ref[:, 1] + 1.0
    hp_ref[...] = h_ref[...] * lax.rsqrt(deg)[:, None]


def _final_body(accp_ref, hp_ref, x2_ref, degp_ref, b_ref, g_ref, be_ref, out_ref):
    deg = degp_ref[:, 0] + degp_ref[:, 1] + 1.0
    dinv = lax.rsqrt(deg)
    acc = accp_ref[0] + accp_ref[1] + hp_ref[...]
    x1 = dinv[:, None] * acc + b_ref[...]
    z = x1 + x2_ref[...] + 1e-6
    mu = jnp.mean(z, axis=-1, keepdims=True)
    zc = z - mu
    var = jnp.mean(zc * zc, axis=-1, keepdims=True)
    out_ref[...] = zc * lax.rsqrt(var + 1e-5) * g_ref[...] + be_ref[...]


def kernel(adj, x, W_gcn, b_gcn, W_lin, gamma, beta):
    n_extra = EPAD - E
    pad_row = jnp.arange(n_extra, dtype=jnp.int32) % N
    pad_col = N + jnp.arange(n_extra, dtype=jnp.int32) % (NPAD - N)
    row = jnp.concatenate([adj[0].astype(jnp.int32), pad_row])
    col = jnp.concatenate([adj[1].astype(jnp.int32), pad_col])
    row = row.reshape(NW, N_CHUNKS, 1, CHUNK)
    col = col.reshape(NW, N_CHUNKS, 1, CHUNK)

    zeros_hist = jnp.zeros((PER_TILE,), jnp.float32)
    ones_chunk = jnp.ones((CHUNK,), jnp.float32)
    zeros_rows = jnp.zeros((PER_TILE, D), jnp.float32)

    degp_full = _deg_kernel(col, zeros_hist, ones_chunk)
    degp = degp_full[:, 0, :N].T

    h_raw, x2 = pl.pallas_call(
        _mm_body,
        grid=(GRID,),
        in_specs=[
            pl.BlockSpec((BLK, D), lambda i: (i, 0)),
            pl.BlockSpec((D, D), lambda i: (0, 0)),
            pl.BlockSpec((D, D), lambda i: (0, 0)),
        ],
        out_specs=[
            pl.BlockSpec((BLK, D), lambda i: (i, 0)),
            pl.BlockSpec((BLK, D), lambda i: (i, 0)),
        ],
        out_shape=[
            jax.ShapeDtypeStruct((N, D), jnp.float32),
            jax.ShapeDtypeStruct((N, D), jnp.float32),
        ],
    )(x, W_gcn.T, W_lin.T)

    hp = pl.pallas_call(
        _scale_body,
        grid=(GRID,),
        in_specs=[
            pl.BlockSpec((BLK, D), lambda i: (i, 0)),
            pl.BlockSpec((BLK, 2), lambda i: (i, 0)),
        ],
        out_specs=pl.BlockSpec((BLK, D), lambda i: (i, 0)),
        out_shape=jax.ShapeDtypeStruct((N, D), jnp.float32),
    )(h_raw, degp)

    accp_full = _edge_kernel(row, col, hp, zeros_rows)

    out = pl.pallas_call(
        _final_body,
        grid=(GRID,),
        in_specs=[
            pl.BlockSpec((2, BLK, D), lambda i: (0, i, 0)),
            pl.BlockSpec((BLK, D), lambda i: (i, 0)),
            pl.BlockSpec((BLK, D), lambda i: (i, 0)),
            pl.BlockSpec((BLK, 2), lambda i: (i, 0)),
            pl.BlockSpec((1, D), lambda i: (0, 0)),
            pl.BlockSpec((1, D), lambda i: (0, 0)),
            pl.BlockSpec((1, D), lambda i: (0, 0)),
        ],
        out_specs=pl.BlockSpec((BLK, D), lambda i: (i, 0)),
        out_shape=jax.ShapeDtypeStruct((N, D), jnp.float32),
    )(accp_full, hp, x2, degp, b_gcn.reshape(1, D), gamma.reshape(1, D), beta.reshape(1, D))

    return out


# speedup vs baseline: 39.9949x; 1.0061x over previous
import functools

import jax
import jax.numpy as jnp
from jax import lax
from jax.experimental import pallas as pl
from jax.experimental.pallas import tpu as pltpu
from jax.experimental.pallas import tpu_sc as plsc

N = 10000
E = 320000
D = 128

NC = 2
NS = 16
NW = NC * NS

CHUNK = 128
EPW = 10240
EPAD = NW * EPW
ROWS_PER_LOAD = 10
N_CHUNKS = EPW // CHUNK
N_LOADS = N_CHUNKS // ROWS_PER_LOAD
DEG_ROWS_PER_LOAD = 5
DEG_N_LOADS = N_CHUNKS // DEG_ROWS_PER_LOAD

NPAD = 10240
PER_TILE = NPAD // NS

_mesh = plsc.VectorSubcoreMesh(core_axis_name="c", subcore_axis_name="s")


@functools.partial(
    pl.kernel,
    out_type=jax.ShapeDtypeStruct((NC, 1, NPAD), jnp.float32),
    mesh=_mesh,
    scratch_types=[
        pltpu.VMEM((DEG_ROWS_PER_LOAD, 1, CHUNK), jnp.int32),
        pltpu.VMEM((CHUNK,), jnp.float32),
        pltpu.VMEM_SHARED((NPAD,), jnp.float32),
        pltpu.SemaphoreType.DMA,
    ],
)
def _deg_kernel(col_hbm, zeros_hbm, ones_hbm, out_hbm, idx_v, ones_v, hist_sp, dsem):
    c = lax.axis_index("c")
    s = lax.axis_index("s")
    wid = c * NS + s
    pltpu.sync_copy(zeros_hbm, hist_sp.at[pl.ds(s * PER_TILE, PER_TILE)])
    pltpu.sync_copy(ones_hbm, ones_v)
    plsc.subcore_barrier()

    def outer(o, _):
        pltpu.sync_copy(
            col_hbm.at[wid, pl.ds(o * DEG_ROWS_PER_LOAD, DEG_ROWS_PER_LOAD)], idx_v
        )
        for j in range(DEG_ROWS_PER_LOAD):
            pltpu.async_copy(ones_v, hist_sp.at[idx_v.at[j, 0]], dsem, add=True)
        for j in range(DEG_ROWS_PER_LOAD):
            pltpu.make_async_copy(ones_v, hist_sp.at[idx_v.at[0, 0]], dsem).wait()
        return 0

    lax.fori_loop(0, DEG_N_LOADS, outer, 0)
    plsc.subcore_barrier()
    pltpu.sync_copy(
        hist_sp.at[pl.ds(s * PER_TILE, PER_TILE)],
        out_hbm.at[c, 0, pl.ds(s * PER_TILE, PER_TILE)],
    )


N_PAIRS = ROWS_PER_LOAD // 2


@functools.partial(
    pl.kernel,
    out_type=jax.ShapeDtypeStruct((NC, NPAD, D), jnp.float32),
    mesh=_mesh,
    scratch_types=[
        pltpu.VMEM((ROWS_PER_LOAD, 1, CHUNK), jnp.int32),
        pltpu.VMEM((ROWS_PER_LOAD, 1, CHUNK), jnp.int32),
        pltpu.VMEM((ROWS_PER_LOAD, 1, CHUNK), jnp.int32),
        pltpu.VMEM((ROWS_PER_LOAD, 1, CHUNK), jnp.int32),
        pltpu.VMEM((CHUNK, D), jnp.float32),
        pltpu.VMEM((CHUNK, D), jnp.float32),
        pltpu.VMEM_SHARED((NPAD, D), jnp.float32),
        pltpu.SemaphoreType.DMA,
        pltpu.SemaphoreType.DMA,
        pltpu.SemaphoreType.DMA,
    ],
)
def _edge_kernel(row_hbm, col_hbm, hp_hbm, zeros_hbm, out_hbm,
                 ir0, ic0, ir1, ic1, b0, b1, acc_sp, sem, isem, ssem):
    c = lax.axis_index("c")
    s = lax.axis_index("s")
    wid = c * NS + s
    d_ir = pltpu.async_copy(row_hbm.at[wid, pl.ds(0, ROWS_PER_LOAD)], ir0, isem)
    d_ic = pltpu.async_copy(col_hbm.at[wid, pl.ds(0, ROWS_PER_LOAD)], ic0, isem)
    d_z = pltpu.async_copy(zeros_hbm, acc_sp.at[pl.ds(s * PER_TILE, PER_TILE)], ssem)
    d_ir.wait()
    d_ic.wait()
    d_z.wait()
    plsc.subcore_barrier()

    def load_block(o, ir, ic):
        base = pl.ds(o * ROWS_PER_LOAD, ROWS_PER_LOAD)
        pltpu.async_copy(row_hbm.at[wid, base], ir, isem)
        pltpu.async_copy(col_hbm.at[wid, base], ic, isem)

    def wait_block(ir, ic):
        pltpu.make_async_copy(row_hbm.at[wid, pl.ds(0, ROWS_PER_LOAD)], ir, isem).wait()
        pltpu.make_async_copy(col_hbm.at[wid, pl.ds(0, ROWS_PER_LOAD)], ic, isem).wait()

    def wait_gather(buf, ir):
        pltpu.make_async_copy(hp_hbm.at[ir.at[0, 0]], buf, sem).wait()

    def wait_scatter(buf, ic):
        pltpu.make_async_copy(buf, acc_sp.at[ic.at[0, 0]], ssem).wait()

    def process_block(ir, ic):
        pltpu.async_copy(hp_hbm.at[ir.at[0, 0]], b0, sem)
        pltpu.async_copy(hp_hbm.at[ir.at[1, 0]], b1, sem)

        def grp(g, _):
            wait_gather(b0, ir)
            pltpu.async_copy(b0, acc_sp.at[ic.at[2 * g, 0]], ssem, add=True)
            wait_gather(b1, ir)
            pltpu.async_copy(b1, acc_sp.at[ic.at[2 * g + 1, 0]], ssem, add=True)
            wait_scatter(b0, ic)
            pltpu.async_copy(hp_hbm.at[ir.at[2 * g + 2, 0]], b0, sem)
            wait_scatter(b1, ic)
            pltpu.async_copy(hp_hbm.at[ir.at[2 * g + 3, 0]], b1, sem)
            return 0

        lax.fori_loop(0, N_PAIRS - 1, grp, 0)
        j = 2 * (N_PAIRS - 1)
        wait_gather(b0, ir)
        pltpu.async_copy(b0, acc_sp.at[ic.at[j, 0]], ssem, add=True)
        wait_gather(b1, ir)
        pltpu.async_copy(b1, acc_sp.at[ic.at[j + 1, 0]], ssem, add=True)
        wait_scatter(b0, ic)
        wait_scatter(b1, ic)

    def outer(o, _):
        even = o % 2 == 0

        @pl.when(jnp.logical_and(even, o < N_LOADS - 1))
        def _():
            load_block(o + 1, ir1, ic1)

        @pl.when(jnp.logical_and(~even, o < N_LOADS - 1))
        def _():
            load_block(o + 1, ir0, ic0)

        @pl.when(even)
        def _():
            process_block(ir0, ic0)

        @pl.when(~even)
        def _():
            process_block(ir1, ic1)

        @pl.when(jnp.logical_and(even, o < N_LOADS - 1))
        def _():
            wait_block(ir1, ic1)

        @pl.when(jnp.logical_and(~even, o < N_LOADS - 1))
        def _():
            wait_block(ir0, ic0)

        return 0

    lax.fori_loop(0, N_LOADS, outer, 0)
    plsc.subcore_barrier()
    pltpu.sync_copy(
        acc_sp.at[pl.ds(s * PER_TILE, PER_TILE)],
        out_hbm.at[c, pl.ds(s * PER_TILE, PER_TILE)],
    )


BLK = 1000
GRID = N // BLK


def _mm_body(x_ref, wg_ref, wl_ref, h_ref, x2_ref):
    h_ref[...] = jnp.dot(x_ref[...], wg_ref[...], preferred_element_type=jnp.float32)
    x2_ref[...] = jnp.dot(x_ref[...], wl_ref[...], preferred_element_type=jnp.float32)


def _scale_body(h_ref, degp_ref, hp_ref):
    deg = degp_ref[:, 0] + degp_ref[:, 1] + 1.0
    hp_ref[...] = h_ref[...] * lax.rsqrt(deg)[:, None]


def _final_body(accp_ref, hp_ref, x2_ref, degp_ref, b_ref, g_ref, be_ref, out_ref):
    deg = degp_ref[:, 0] + degp_ref[:, 1] + 1.0
    dinv = lax.rsqrt(deg)
    acc = accp_ref[0] + accp_ref[1] + hp_ref[...]
    x1 = dinv[:, None] * acc + b_ref[...]
    z = x1 + x2_ref[...] + 1e-6
    mu = jnp.mean(z, axis=-1, keepdims=True)
    zc = z - mu
    var = jnp.mean(zc * zc, axis=-1, keepdims=True)
    out_ref[...] = zc * lax.rsqrt(var + 1e-5) * g_ref[...] + be_ref[...]


def kernel(adj, x, W_gcn, b_gcn, W_lin, gamma, beta):
    n_extra = EPAD - E
    pad_row = jnp.arange(n_extra, dtype=jnp.int32) % N
    pad_col = N + jnp.arange(n_extra, dtype=jnp.int32) % (NPAD - N)
    row = jnp.concatenate([adj[0].astype(jnp.int32), pad_row])
    col = jnp.concatenate([adj[1].astype(jnp.int32), pad_col])
    row = row.reshape(NW, N_CHUNKS, 1, CHUNK)
    col = col.reshape(NW, N_CHUNKS, 1, CHUNK)

    zeros_hist = jnp.zeros((PER_TILE,), jnp.float32)
    ones_chunk = jnp.ones((CHUNK,), jnp.float32)
    zeros_rows = jnp.zeros((PER_TILE, D), jnp.float32)

    degp_full = _deg_kernel(col, zeros_hist, ones_chunk)
    degp = degp_full[:, 0, :N].T

    h_raw, x2 = pl.pallas_call(
        _mm_body,
        grid=(GRID,),
        in_specs=[
            pl.BlockSpec((BLK, D), lambda i: (i, 0)),
            pl.BlockSpec((D, D), lambda i: (0, 0)),
            pl.BlockSpec((D, D), lambda i: (0, 0)),
        ],
        out_specs=[
            pl.BlockSpec((BLK, D), lambda i: (i, 0)),
            pl.BlockSpec((BLK, D), lambda i: (i, 0)),
        ],
        out_shape=[
            jax.ShapeDtypeStruct((N, D), jnp.float32),
            jax.ShapeDtypeStruct((N, D), jnp.float32),
        ],
    )(x, W_gcn.T, W_lin.T)

    hp = pl.pallas_call(
        _scale_body,
        grid=(GRID,),
        in_specs=[
            pl.BlockSpec((BLK, D), lambda i: (i, 0)),
            pl.BlockSpec((BLK, 2), lambda i: (i, 0)),
        ],
        out_specs=pl.BlockSpec((BLK, D), lambda i: (i, 0)),
        out_shape=jax.ShapeDtypeStruct((N, D), jnp.float32),
    )(h_raw, degp)

    accp_full = _edge_kernel(row, col, hp, zeros_rows)

    out = pl.pallas_call(
        _final_body,
        grid=(GRID,),
        in_specs=[
            pl.BlockSpec((2, BLK, D), lambda i: (0, i, 0)),
            pl.BlockSpec((BLK, D), lambda i: (i, 0)),
            pl.BlockSpec((BLK, D), lambda i: (i, 0)),
            pl.BlockSpec((BLK, 2), lambda i: (i, 0)),
            pl.BlockSpec((1, D), lambda i: (0, 0)),
            pl.BlockSpec((1, D), lambda i: (0, 0)),
            pl.BlockSpec((1, D), lambda i: (0, 0)),
        ],
        out_specs=pl.BlockSpec((BLK, D), lambda i: (i, 0)),
        out_shape=jax.ShapeDtypeStruct((N, D), jnp.float32),
    )(accp_full, hp, x2, degp, b_gcn.reshape(1, D), gamma.reshape(1, D), beta.reshape(1, D))

    return out


# speedup vs baseline: 40.5084x; 1.0128x over previous
import functools

import jax
import jax.numpy as jnp
from jax import lax
from jax.experimental import pallas as pl
from jax.experimental.pallas import tpu as pltpu
from jax.experimental.pallas import tpu_sc as plsc

N = 10000
E = 320000
D = 128

NC = 2
NS = 16
NW = NC * NS

CHUNK = 128
EPW = 10240
EPAD = NW * EPW
ROWS_PER_LOAD = 10
N_CHUNKS = EPW // CHUNK
N_LOADS = N_CHUNKS // ROWS_PER_LOAD
DEG_ROWS_PER_LOAD = 5
DEG_N_LOADS = N_CHUNKS // DEG_ROWS_PER_LOAD

NPAD = 10240
PER_TILE = NPAD // NS

_mesh = plsc.VectorSubcoreMesh(core_axis_name="c", subcore_axis_name="s")


@functools.partial(
    pl.kernel,
    out_type=jax.ShapeDtypeStruct((NC, 1, NPAD), jnp.float32),
    mesh=_mesh,
    scratch_types=[
        pltpu.VMEM((DEG_ROWS_PER_LOAD, 1, CHUNK), jnp.int32),
        pltpu.VMEM((CHUNK,), jnp.float32),
        pltpu.VMEM_SHARED((NPAD,), jnp.float32),
        pltpu.SemaphoreType.DMA,
    ],
)
def _deg_kernel(col_hbm, zeros_hbm, ones_hbm, out_hbm, idx_v, ones_v, hist_sp, dsem):
    c = lax.axis_index("c")
    s = lax.axis_index("s")
    wid = c * NS + s
    d_z = pltpu.async_copy(zeros_hbm, hist_sp.at[pl.ds(s * PER_TILE, PER_TILE)], dsem)
    d_o = pltpu.async_copy(ones_hbm, ones_v, dsem)
    d_z.wait()
    d_o.wait()
    plsc.subcore_barrier()

    def outer(o, _):
        pltpu.sync_copy(
            col_hbm.at[wid, pl.ds(o * DEG_ROWS_PER_LOAD, DEG_ROWS_PER_LOAD)], idx_v
        )
        for j in range(DEG_ROWS_PER_LOAD):
            pltpu.async_copy(ones_v, hist_sp.at[idx_v.at[j, 0]], dsem, add=True)
        for j in range(DEG_ROWS_PER_LOAD):
            pltpu.make_async_copy(ones_v, hist_sp.at[idx_v.at[0, 0]], dsem).wait()
        return 0

    lax.fori_loop(0, DEG_N_LOADS, outer, 0)
    plsc.subcore_barrier()
    pltpu.sync_copy(
        hist_sp.at[pl.ds(s * PER_TILE, PER_TILE)],
        out_hbm.at[c, 0, pl.ds(s * PER_TILE, PER_TILE)],
    )


N_PAIRS = ROWS_PER_LOAD // 2


@functools.partial(
    pl.kernel,
    out_type=jax.ShapeDtypeStruct((NC, NPAD, D), jnp.float32),
    mesh=_mesh,
    scratch_types=[
        pltpu.VMEM((ROWS_PER_LOAD, 1, CHUNK), jnp.int32),
        pltpu.VMEM((ROWS_PER_LOAD, 1, CHUNK), jnp.int32),
        pltpu.VMEM((ROWS_PER_LOAD, 1, CHUNK), jnp.int32),
        pltpu.VMEM((ROWS_PER_LOAD, 1, CHUNK), jnp.int32),
        pltpu.VMEM((CHUNK, D), jnp.float32),
        pltpu.VMEM((CHUNK, D), jnp.float32),
        pltpu.VMEM_SHARED((NPAD, D), jnp.float32),
        pltpu.SemaphoreType.DMA,
        pltpu.SemaphoreType.DMA,
        pltpu.SemaphoreType.DMA,
    ],
)
def _edge_kernel(row_hbm, col_hbm, hp_hbm, zeros_hbm, out_hbm,
                 ir0, ic0, ir1, ic1, b0, b1, acc_sp, sem, isem, ssem):
    c = lax.axis_index("c")
    s = lax.axis_index("s")
    wid = c * NS + s
    d_ir = pltpu.async_copy(row_hbm.at[wid, pl.ds(0, ROWS_PER_LOAD)], ir0, isem)
    d_ic = pltpu.async_copy(col_hbm.at[wid, pl.ds(0, ROWS_PER_LOAD)], ic0, isem)
    d_z = pltpu.async_copy(zeros_hbm, acc_sp.at[pl.ds(s * PER_TILE, PER_TILE)], ssem)
    d_ir.wait()
    d_ic.wait()
    d_z.wait()
    plsc.subcore_barrier()

    def load_block(o, ir, ic):
        base = pl.ds(o * ROWS_PER_LOAD, ROWS_PER_LOAD)
        pltpu.async_copy(row_hbm.at[wid, base], ir, isem)
        pltpu.async_copy(col_hbm.at[wid, base], ic, isem)

    def wait_block(ir, ic):
        pltpu.make_async_copy(row_hbm.at[wid, pl.ds(0, ROWS_PER_LOAD)], ir, isem).wait()
        pltpu.make_async_copy(col_hbm.at[wid, pl.ds(0, ROWS_PER_LOAD)], ic, isem).wait()

    def wait_gather(buf, ir):
        pltpu.make_async_copy(hp_hbm.at[ir.at[0, 0]], buf, sem).wait()

    def wait_scatter(buf, ic):
        pltpu.make_async_copy(buf, acc_sp.at[ic.at[0, 0]], ssem).wait()

    def process_block(ir, ic):
        pltpu.async_copy(hp_hbm.at[ir.at[0, 0]], b0, sem)
        pltpu.async_copy(hp_hbm.at[ir.at[1, 0]], b1, sem)

        def grp(g, _):
            wait_gather(b0, ir)
            pltpu.async_copy(b0, acc_sp.at[ic.at[2 * g, 0]], ssem, add=True)
            wait_gather(b1, ir)
            pltpu.async_copy(b1, acc_sp.at[ic.at[2 * g + 1, 0]], ssem, add=True)
            wait_scatter(b0, ic)
            pltpu.async_copy(hp_hbm.at[ir.at[2 * g + 2, 0]], b0, sem)
            wait_scatter(b1, ic)
            pltpu.async_copy(hp_hbm.at[ir.at[2 * g + 3, 0]], b1, sem)
            return 0

        lax.fori_loop(0, N_PAIRS - 1, grp, 0)
        j = 2 * (N_PAIRS - 1)
        wait_gather(b0, ir)
        pltpu.async_copy(b0, acc_sp.at[ic.at[j, 0]], ssem, add=True)
        wait_gather(b1, ir)
        pltpu.async_copy(b1, acc_sp.at[ic.at[j + 1, 0]], ssem, add=True)
        wait_scatter(b0, ic)
        wait_scatter(b1, ic)

    def outer(o, _):
        even = o % 2 == 0

        @pl.when(jnp.logical_and(even, o < N_LOADS - 1))
        def _():
            load_block(o + 1, ir1, ic1)

        @pl.when(jnp.logical_and(~even, o < N_LOADS - 1))
        def _():
            load_block(o + 1, ir0, ic0)

        @pl.when(even)
        def _():
            process_block(ir0, ic0)

        @pl.when(~even)
        def _():
            process_block(ir1, ic1)

        @pl.when(jnp.logical_and(even, o < N_LOADS - 1))
        def _():
            wait_block(ir1, ic1)

        @pl.when(jnp.logical_and(~even, o < N_LOADS - 1))
        def _():
            wait_block(ir0, ic0)

        return 0

    lax.fori_loop(0, N_LOADS, outer, 0)
    plsc.subcore_barrier()
    pltpu.sync_copy(
        acc_sp.at[pl.ds(s * PER_TILE, PER_TILE)],
        out_hbm.at[c, pl.ds(s * PER_TILE, PER_TILE)],
    )


BLK = 1000
GRID = N // BLK


def _mm_body(x_ref, wg_ref, wl_ref, h_ref, x2_ref):
    h_ref[...] = jnp.dot(x_ref[...], wg_ref[...], preferred_element_type=jnp.float32)
    x2_ref[...] = jnp.dot(x_ref[...], wl_ref[...], preferred_element_type=jnp.float32)


def _scale_body(h_ref, degp_ref, hp_ref):
    deg = degp_ref[:, 0] + degp_ref[:, 1] + 1.0
    hp_ref[...] = h_ref[...] * lax.rsqrt(deg)[:, None]


def _final_body(accp_ref, hp_ref, x2_ref, degp_ref, b_ref, g_ref, be_ref, out_ref):
    deg = degp_ref[:, 0] + degp_ref[:, 1] + 1.0
    dinv = lax.rsqrt(deg)
    acc = accp_ref[0] + accp_ref[1] + hp_ref[...]
    x1 = dinv[:, None] * acc + b_ref[...]
    z = x1 + x2_ref[...] + 1e-6
    mu = jnp.mean(z, axis=-1, keepdims=True)
    zc = z - mu
    var = jnp.mean(zc * zc, axis=-1, keepdims=True)
    out_ref[...] = zc * lax.rsqrt(var + 1e-5) * g_ref[...] + be_ref[...]


def kernel(adj, x, W_gcn, b_gcn, W_lin, gamma, beta):
    n_extra = EPAD - E
    pad_row = jnp.arange(n_extra, dtype=jnp.int32) % N
    pad_col = N + jnp.arange(n_extra, dtype=jnp.int32) % (NPAD - N)
    row = jnp.concatenate([adj[0].astype(jnp.int32), pad_row])
    col = jnp.concatenate([adj[1].astype(jnp.int32), pad_col])
    row = row.reshape(NW, N_CHUNKS, 1, CHUNK)
    col = col.reshape(NW, N_CHUNKS, 1, CHUNK)

    zeros_hist = jnp.zeros((PER_TILE,), jnp.float32)
    ones_chunk = jnp.ones((CHUNK,), jnp.float32)
    zeros_rows = jnp.zeros((PER_TILE, D), jnp.float32)

    degp_full = _deg_kernel(col, zeros_hist, ones_chunk)
    degp = degp_full[:, 0, :N].T

    h_raw, x2 = pl.pallas_call(
        _mm_body,
        grid=(GRID,),
        in_specs=[
            pl.BlockSpec((BLK, D), lambda i: (i, 0)),
            pl.BlockSpec((D, D), lambda i: (0, 0)),
            pl.BlockSpec((D, D), lambda i: (0, 0)),
        ],
        out_specs=[
            pl.BlockSpec((BLK, D), lambda i: (i, 0)),
            pl.BlockSpec((BLK, D), lambda i: (i, 0)),
        ],
        out_shape=[
            jax.ShapeDtypeStruct((N, D), jnp.float32),
            jax.ShapeDtypeStruct((N, D), jnp.float32),
        ],
    )(x, W_gcn.T, W_lin.T)

    hp = pl.pallas_call(
        _scale_body,
        grid=(GRID,),
        in_specs=[
            pl.BlockSpec((BLK, D), lambda i: (i, 0)),
            pl.BlockSpec((BLK, 2), lambda i: (i, 0)),
        ],
        out_specs=pl.BlockSpec((BLK, D), lambda i: (i, 0)),
        out_shape=jax.ShapeDtypeStruct((N, D), jnp.float32),
    )(h_raw, degp)

    accp_full = _edge_kernel(row, col, hp, zeros_rows)

    out = pl.pallas_call(
        _final_body,
        grid=(GRID,),
        in_specs=[
            pl.BlockSpec((2, BLK, D), lambda i: (0, i, 0)),
            pl.BlockSpec((BLK, D), lambda i: (i, 0)),
            pl.BlockSpec((BLK, D), lambda i: (i, 0)),
            pl.BlockSpec((BLK, 2), lambda i: (i, 0)),
            pl.BlockSpec((1, D), lambda i: (0, 0)),
            pl.BlockSpec((1, D), lambda i: (0, 0)),
            pl.BlockSpec((1, D), lambda i: (0, 0)),
        ],
        out_specs=pl.BlockSpec((BLK, D), lambda i: (i, 0)),
        out_shape=jax.ShapeDtypeStruct((N, D), jnp.float32),
    )(accp_full, hp, x2, degp, b_gcn.reshape(1, D), gamma.reshape(1, D), beta.reshape(1, D))

    return out
